# baseline (speedup 1.0000x reference)
import functools

import jax
import jax.numpy as jnp
from jax import lax
from jax.experimental import pallas as pl

DIN = 128
A = 4
H = 11
W = 11
HW = H * W
N = HW * A
NP = 512
TOPN = 300
BLK1 = 128
BLK2 = 8

_WA = (5.0, 9.0, 5.0, 7.0)
_HA = (5.0, 9.0, 9.0, 9.0)


def _stage1_body(feat_ref, w_ref, bias_ref, gt_ref, imf_ref,
                 t_ref, x1_ref, y1_ref, x2_ref, y2_ref, logp_ref):
    feat = feat_ref[...]
    wcat = w_ref[...]
    bias = bias_ref[...]
    wb = jnp.broadcast_to(wcat[None], (BLK1, 24, DIN))
    outt = lax.dot_general(wb, feat, (((2,), (1,)), ((0,), (0,))),
                           preferred_element_type=jnp.float32)
    outt = outt + bias[None]

    s1 = outt[:, 0:4, :]
    s0 = outt[:, 4:8, :]
    dx = outt[:, 8:12, :]
    dy = outt[:, 12:16, :]
    dw = outt[:, 16:20, :]
    dh = outt[:, 20:24, :]
    t = s1 - s0

    et = jnp.exp(-jnp.abs(t))
    logp = jnp.where(t >= 0.0, -jnp.log(1.0 + et), t - jnp.log(1.0 + et))

    p = lax.broadcasted_iota(jnp.int32, (A, HW), 1)
    ai = lax.broadcasted_iota(jnp.int32, (A, HW), 0)
    gx = (p % W).astype(jnp.float32)
    gy = (p // W).astype(jnp.float32)
    av = lax.broadcasted_iota(jnp.int32, (A, 1), 0)
    wa = jnp.where(av == 1, 9.0, jnp.where(av == 3, 7.0, 5.0))
    ha = jnp.where(av == 0, 5.0, 9.0)
    ctr_x = gx + 0.5 * (wa - 1.0)
    ctr_y = gy + 0.5 * (ha - 1.0)

    pcx = dx * wa + ctr_x[None]
    pcy = dy * ha + ctr_y[None]
    pw = jnp.exp(jnp.clip(dw, -10.0, 10.0)) * wa
    ph = jnp.exp(jnp.clip(dh, -10.0, 10.0)) * ha
    imf = imf_ref[0, 0]
    x1 = jnp.clip(pcx - 0.5 * (pw - 1.0), 0.0, imf)
    y1 = jnp.clip(pcy - 0.5 * (ph - 1.0), 0.0, imf)
    x2 = jnp.clip(pcx + 0.5 * (pw - 1.0), 0.0, imf)
    y2 = jnp.clip(pcy + 0.5 * (ph - 1.0), 0.0, imf)

    gt = gt_ref[...]
    gcx = 0.5 * (gt[:, 0:1] + gt[:, 2:3])
    gcy = 0.5 * (gt[:, 1:2] + gt[:, 3:4])
    ddx = ctr_x[None] - gcx[:, :, None]
    ddy = ctr_y[None] - gcy[:, :, None]
    dist = ddx * ddx + ddy * ddy
    n_arr = 4 * p + ai
    dmin = jnp.min(dist, axis=(1, 2), keepdims=True)
    nsel = jnp.min(jnp.where(dist == dmin, n_arr[None], jnp.int32(1 << 30)),
                   axis=(1, 2), keepdims=True)
    lsel = jnp.sum(jnp.where(n_arr[None] == nsel, logp, 0.0), axis=(1, 2))

    padt = jnp.full((BLK1, A, 128 - HW), -1e30, dtype=jnp.float32)
    pad0 = jnp.zeros((BLK1, A, 128 - HW), dtype=jnp.float32)
    t_ref[...] = jnp.concatenate([t, padt], axis=-1)
    x1_ref[...] = jnp.concatenate([x1, pad0], axis=-1)
    y1_ref[...] = jnp.concatenate([y1, pad0], axis=-1)
    x2_ref[...] = jnp.concatenate([x2, pad0], axis=-1)
    y2_ref[...] = jnp.concatenate([y2, pad0], axis=-1)
    logp_ref[...] = jnp.broadcast_to(lsel[:, None], (BLK1, 128))


def _stage2_body(t_ref, x1_ref, y1_ref, x2_ref, y2_ref,
                 logp_ref, cp_ref, gt_ref,
                 sorted_ref, loss_ref):
    nb = pl.num_programs(0)
    t = t_ref[...]
    ii = lax.broadcasted_iota(jnp.int32, (NP, NP), 0)
    jj = lax.broadcasted_iota(jnp.int32, (NP, NP), 1)
    tri = jj < ii
    gt_m = t[:, None, :] > t[:, :, None]
    eq_m = t[:, None, :] == t[:, :, None]
    cmp = jnp.where(gt_m | (eq_m & tri[None]), 1.0, 0.0)
    rank = jnp.sum(cmp, axis=2).astype(jnp.int32)
    rr = lax.broadcasted_iota(jnp.int32, (1, 1, NP), 2)
    perm = jnp.where(rank[:, :, None] == rr, 1.0, 0.0)

    vals = jnp.concatenate(
        [t[:, None, :], x1_ref[...][:, None, :], y1_ref[...][:, None, :],
         x2_ref[...][:, None, :], y2_ref[...][:, None, :]], axis=1)
    svals = lax.dot_general(vals, perm, (((2,), (1,)), ((0,), (0,))),
                            preferred_element_type=jnp.float32)
    st = svals[:, 0:1, :]
    et = jnp.exp(-jnp.abs(st))
    score = jnp.where(st >= 0.0, 1.0 / (1.0 + et), et / (1.0 + et))
    sorted_ref[...] = jnp.concatenate([score, svals[:, 1:5, :]], axis=1)

    cp = cp_ref[...]
    rri = lax.broadcasted_iota(jnp.int32, (BLK2, NP), 1)
    msk = jnp.where(rri == cp, 1.0, 0.0)
    pb = jnp.sum(svals[:, 1:5, :] * msk[:, None, :], axis=2)
    d = pb - gt_ref[...]
    ad = jnp.abs(d)
    sl1 = jnp.sum(jnp.where(ad < 3.0, d * d, ad))
    lcls = jnp.sum(logp_ref[...][:, 0])

    part = sl1 / (4.0 * BLK2 * nb) - lcls / (BLK2 * nb)

    @pl.when(pl.program_id(0) == 0)
    def _():
        loss_ref[...] = jnp.zeros((1, 128), dtype=jnp.float32)

    loss_ref[...] += jnp.broadcast_to(part, (1, 128))


@jax.jit
def _run(feat3, wcat, bias, gtb, imf, cp):
    b = feat3.shape[0]
    g1 = b // BLK1
    outs1 = pl.pallas_call(
        _stage1_body,
        grid=(g1,),
        in_specs=[
            pl.BlockSpec((BLK1, DIN, HW), lambda i: (i, 0, 0)),
            pl.BlockSpec((24, DIN), lambda i: (0, 0)),
            pl.BlockSpec((24, HW), lambda i: (0, 0)),
            pl.BlockSpec((BLK1, 4), lambda i: (i, 0)),
            pl.BlockSpec((8, 128), lambda i: (0, 0)),
        ],
        out_specs=[
            pl.BlockSpec((BLK1, A, 128), lambda i: (i, 0, 0)),
            pl.BlockSpec((BLK1, A, 128), lambda i: (i, 0, 0)),
            pl.BlockSpec((BLK1, A, 128), lambda i: (i, 0, 0)),
            pl.BlockSpec((BLK1, A, 128), lambda i: (i, 0, 0)),
            pl.BlockSpec((BLK1, A, 128), lambda i: (i, 0, 0)),
            pl.BlockSpec((BLK1, 128), lambda i: (i, 0)),
        ],
        out_shape=[jax.ShapeDtypeStruct((b, A, 128), jnp.float32)] * 5
        + [jax.ShapeDtypeStruct((b, 128), jnp.float32)],
    )(feat3, wcat, bias, gtb, imf)
    t5, x15, y15, x25, y25, logp = outs1
    t512 = t5.reshape(b, NP)
    x1p = x15.reshape(b, NP)
    y1p = y15.reshape(b, NP)
    x2p = x25.reshape(b, NP)
    y2p = y25.reshape(b, NP)

    g2 = b // BLK2
    sorted5, loss = pl.pallas_call(
        _stage2_body,
        grid=(g2,),
        in_specs=[
            pl.BlockSpec((BLK2, NP), lambda i: (i, 0)),
            pl.BlockSpec((BLK2, NP), lambda i: (i, 0)),
            pl.BlockSpec((BLK2, NP), lambda i: (i, 0)),
            pl.BlockSpec((BLK2, NP), lambda i: (i, 0)),
            pl.BlockSpec((BLK2, NP), lambda i: (i, 0)),
            pl.BlockSpec((BLK2, 128), lambda i: (i, 0)),
            pl.BlockSpec((BLK2, 1), lambda i: (i, 0)),
            pl.BlockSpec((BLK2, 4), lambda i: (i, 0)),
        ],
        out_specs=[
            pl.BlockSpec((BLK2, 5, NP), lambda i: (i, 0, 0)),
            pl.BlockSpec((1, 128), lambda i: (0, 0)),
        ],
        out_shape=[
            jax.ShapeDtypeStruct((b, 5, NP), jnp.float32),
            jax.ShapeDtypeStruct((1, 128), jnp.float32),
        ],
    )(t512, x1p, y1p, x2p, y2p, logp, cp, gtb)
    return sorted5, loss


def kernel(base_feat, central_pos, im_info, gt_boxes, W_cls, b_cls, W_bbox, b_bbox):
    b = base_feat.shape[0]
    feat3 = base_feat.reshape(b, DIN, HW)
    wcat = jnp.concatenate([W_cls[1::2], W_cls[0::2], W_bbox[0::4],
                            W_bbox[1::4], W_bbox[2::4], W_bbox[3::4]], axis=0)
    bcat = jnp.concatenate([b_cls[1::2], b_cls[0::2], b_bbox[0::4],
                            b_bbox[1::4], b_bbox[2::4], b_bbox[3::4]], axis=0)
    bias = jnp.broadcast_to(bcat[:, None], (24, HW))
    imf = jnp.full((8, 128), jnp.float32(im_info), dtype=jnp.float32)
    cp = central_pos.astype(jnp.int32).reshape(b, 1)

    sorted5, loss = _run(feat3, wcat, bias, gt_boxes, imf, cp)

    topv = sorted5[:, 0, :TOPN]
    boxes = jnp.transpose(sorted5[:, 1:5, :TOPN], (0, 2, 1))
    bidx = jnp.broadcast_to(
        jnp.arange(b, dtype=jnp.float32)[:, None, None], (b, TOPN, 1))
    pad = jnp.zeros((b, TOPN, 2), dtype=jnp.float32)
    output = jnp.concatenate([bidx, topv[..., None], pad, boxes],
                             axis=-1).reshape(b * TOPN, 8)
    return (output, loss[0, 0].reshape(()))


# speedup vs baseline: 2.1005x; 2.1005x over previous
import functools

import jax
import jax.numpy as jnp
from jax import lax
from jax.experimental import pallas as pl
from jax.experimental.pallas import tpu as pltpu

DIN = 128
A = 4
H = 11
W = 11
HW = H * W
N = HW * A
NP = 512
TOPN = 300
BLK1 = 128
BLK2 = 64

_WA = (5.0, 9.0, 5.0, 7.0)
_HA = (5.0, 9.0, 9.0, 9.0)


def _stage1_body(feat_ref, w_ref, bias_ref, gt_ref, imf_ref,
                 t_ref, x1_ref, y1_ref, x2_ref, y2_ref, logp_ref):
    feat = feat_ref[...]
    wcat = w_ref[...]
    bias = bias_ref[...]
    wb = jnp.broadcast_to(wcat[None], (BLK1, 24, DIN))
    outt = lax.dot_general(wb, feat, (((2,), (1,)), ((0,), (0,))),
                           preferred_element_type=jnp.float32)
    outt = outt + bias[None]

    s1 = outt[:, 0:4, :]
    s0 = outt[:, 4:8, :]
    dx = outt[:, 8:12, :]
    dy = outt[:, 12:16, :]
    dw = outt[:, 16:20, :]
    dh = outt[:, 20:24, :]
    t = s1 - s0

    et = jnp.exp(-jnp.abs(t))
    logp = jnp.where(t >= 0.0, -jnp.log(1.0 + et), t - jnp.log(1.0 + et))

    p = lax.broadcasted_iota(jnp.int32, (A, HW), 1)
    ai = lax.broadcasted_iota(jnp.int32, (A, HW), 0)
    gx = (p % W).astype(jnp.float32)
    gy = (p // W).astype(jnp.float32)
    av = lax.broadcasted_iota(jnp.int32, (A, 1), 0)
    wa = jnp.where(av == 1, 9.0, jnp.where(av == 3, 7.0, 5.0))
    ha = jnp.where(av == 0, 5.0, 9.0)
    ctr_x = gx + 0.5 * (wa - 1.0)
    ctr_y = gy + 0.5 * (ha - 1.0)

    pcx = dx * wa + ctr_x[None]
    pcy = dy * ha + ctr_y[None]
    pw = jnp.exp(jnp.clip(dw, -10.0, 10.0)) * wa
    ph = jnp.exp(jnp.clip(dh, -10.0, 10.0)) * ha
    imf = imf_ref[0, 0]
    x1 = jnp.clip(pcx - 0.5 * (pw - 1.0), 0.0, imf)
    y1 = jnp.clip(pcy - 0.5 * (ph - 1.0), 0.0, imf)
    x2 = jnp.clip(pcx + 0.5 * (pw - 1.0), 0.0, imf)
    y2 = jnp.clip(pcy + 0.5 * (ph - 1.0), 0.0, imf)

    gt = gt_ref[...]
    gcx = 0.5 * (gt[:, 0:1] + gt[:, 2:3])
    gcy = 0.5 * (gt[:, 1:2] + gt[:, 3:4])
    ddx = ctr_x[None] - gcx[:, :, None]
    ddy = ctr_y[None] - gcy[:, :, None]
    dist = ddx * ddx + ddy * ddy
    n_arr = 4 * p + ai
    dmin = jnp.min(dist, axis=(1, 2), keepdims=True)
    nsel = jnp.min(jnp.where(dist == dmin, n_arr[None], jnp.int32(1 << 30)),
                   axis=(1, 2), keepdims=True)
    lsel = jnp.sum(jnp.where(n_arr[None] == nsel, logp, 0.0), axis=(1, 2))

    padt = jnp.full((BLK1, A, 128 - HW), -1e30, dtype=jnp.float32)
    pad0 = jnp.zeros((BLK1, A, 128 - HW), dtype=jnp.float32)
    t_ref[...] = jnp.concatenate([t, padt], axis=-1)
    x1_ref[...] = jnp.concatenate([x1, pad0], axis=-1)
    y1_ref[...] = jnp.concatenate([y1, pad0], axis=-1)
    x2_ref[...] = jnp.concatenate([x2, pad0], axis=-1)
    y2_ref[...] = jnp.concatenate([y2, pad0], axis=-1)
    logp_ref[...] = jnp.broadcast_to(lsel[:, None], (BLK1, 128))


def _stage2_body(t_ref, x1_ref, y1_ref, x2_ref, y2_ref,
                 logp_ref, cp_ref, gt_ref,
                 sorted_ref, loss_ref):
    nb = pl.num_programs(0)
    key = t_ref[...]
    il = lax.broadcasted_iota(jnp.int32, (1, NP), 1)
    idx = jnp.broadcast_to(il, (BLK2, NP))

    for k in range(1, 10):
        blk_desc = ((il >> k) & 1) == 0
        for j in range(k - 1, -1, -1):
            d = 1 << j
            bit0 = ((il >> j) & 1) == 0
            take_max = blk_desc == bit0
            km = pltpu.roll(key, NP - d, 1)
            kpl = pltpu.roll(key, d, 1)
            kp = jnp.where(bit0, km, kpl)
            im = pltpu.roll(idx, NP - d, 1)
            ipl = pltpu.roll(idx, d, 1)
            ip = jnp.where(bit0, im, ipl)
            keep_self = take_max == (key > kp)
            key = jnp.where(keep_self, key, kp)
            idx = jnp.where(keep_self, idx, ip)

    st = key[:, None, :]
    et = jnp.exp(-jnp.abs(st))
    score = jnp.where(st >= 0.0, 1.0 / (1.0 + et), et / (1.0 + et))
    idxm = idx & 127
    hi = idx >> 7

    def _gather512(src):
        out = None
        for c in range(4):
            g = jnp.take_along_axis(src[:, c * 128:(c + 1) * 128], idxm,
                                    axis=1)
            out = g if out is None else jnp.where(hi == c, g, out)
        return out

    gathered = [_gather512(r[...]) for r in (x1_ref, y1_ref, x2_ref, y2_ref)]
    svals = jnp.concatenate([a[:, None, :] for a in gathered], axis=1)
    sorted_ref[...] = jnp.concatenate([score, svals], axis=1)

    cp = cp_ref[...]
    msk = il == cp
    gt = gt_ref[...]
    sl1 = jnp.float32(0.0)
    for c in range(4):
        pb = jnp.sum(jnp.where(msk, gathered[c], 0.0), axis=1)
        d = pb - gt[:, c]
        ad = jnp.abs(d)
        sl1 += jnp.sum(jnp.where(ad < 3.0, d * d, ad))
    lcls = jnp.sum(logp_ref[...][:, 0])

    part = sl1 / (4.0 * BLK2 * nb) - lcls / (BLK2 * nb)

    @pl.when(pl.program_id(0) == 0)
    def _():
        loss_ref[...] = jnp.zeros((1, 128), dtype=jnp.float32)

    loss_ref[...] += jnp.broadcast_to(part, (1, 128))


@jax.jit
def _run(feat3, wcat, bias, gtb, imf, cp):
    b = feat3.shape[0]
    g1 = b // BLK1
    outs1 = pl.pallas_call(
        _stage1_body,
        grid=(g1,),
        in_specs=[
            pl.BlockSpec((BLK1, DIN, HW), lambda i: (i, 0, 0)),
            pl.BlockSpec((24, DIN), lambda i: (0, 0)),
            pl.BlockSpec((24, HW), lambda i: (0, 0)),
            pl.BlockSpec((BLK1, 4), lambda i: (i, 0)),
            pl.BlockSpec((8, 128), lambda i: (0, 0)),
        ],
        out_specs=[
            pl.BlockSpec((BLK1, A, 128), lambda i: (i, 0, 0)),
            pl.BlockSpec((BLK1, A, 128), lambda i: (i, 0, 0)),
            pl.BlockSpec((BLK1, A, 128), lambda i: (i, 0, 0)),
            pl.BlockSpec((BLK1, A, 128), lambda i: (i, 0, 0)),
            pl.BlockSpec((BLK1, A, 128), lambda i: (i, 0, 0)),
            pl.BlockSpec((BLK1, 128), lambda i: (i, 0)),
        ],
        out_shape=[jax.ShapeDtypeStruct((b, A, 128), jnp.float32)] * 5
        + [jax.ShapeDtypeStruct((b, 128), jnp.float32)],
    )(feat3, wcat, bias, gtb, imf)
    t5, x15, y15, x25, y25, logp = outs1

    g2 = b // BLK2
    sorted5, loss = pl.pallas_call(
        _stage2_body,
        grid=(g2,),
        in_specs=[
            pl.BlockSpec((BLK2, NP), lambda i: (i, 0)),
            pl.BlockSpec((BLK2, NP), lambda i: (i, 0)),
            pl.BlockSpec((BLK2, NP), lambda i: (i, 0)),
            pl.BlockSpec((BLK2, NP), lambda i: (i, 0)),
            pl.BlockSpec((BLK2, NP), lambda i: (i, 0)),
            pl.BlockSpec((BLK2, 128), lambda i: (i, 0)),
            pl.BlockSpec((BLK2, 1), lambda i: (i, 0)),
            pl.BlockSpec((BLK2, 4), lambda i: (i, 0)),
        ],
        out_specs=[
            pl.BlockSpec((BLK2, 5, NP), lambda i: (i, 0, 0)),
            pl.BlockSpec((1, 128), lambda i: (0, 0)),
        ],
        out_shape=[
            jax.ShapeDtypeStruct((b, 5, NP), jnp.float32),
            jax.ShapeDtypeStruct((1, 128), jnp.float32),
        ],
    )(t5.reshape(b, NP), x15.reshape(b, NP), y15.reshape(b, NP),
      x25.reshape(b, NP), y25.reshape(b, NP), logp, cp, gtb)
    return sorted5, loss


def kernel(base_feat, central_pos, im_info, gt_boxes, W_cls, b_cls, W_bbox, b_bbox):
    b = base_feat.shape[0]
    feat3 = base_feat.reshape(b, DIN, HW)
    wcat = jnp.concatenate([W_cls[1::2], W_cls[0::2], W_bbox[0::4],
                            W_bbox[1::4], W_bbox[2::4], W_bbox[3::4]], axis=0)
    bcat = jnp.concatenate([b_cls[1::2], b_cls[0::2], b_bbox[0::4],
                            b_bbox[1::4], b_bbox[2::4], b_bbox[3::4]], axis=0)
    bias = jnp.broadcast_to(bcat[:, None], (24, HW))
    imf = jnp.full((8, 128), jnp.float32(im_info), dtype=jnp.float32)
    cp = central_pos.astype(jnp.int32).reshape(b, 1)

    sorted5, loss = _run(feat3, wcat, bias, gt_boxes, imf, cp)

    topv = sorted5[:, 0, :TOPN]
    boxes = jnp.transpose(sorted5[:, 1:5, :TOPN], (0, 2, 1))
    bidx = jnp.broadcast_to(
        jnp.arange(b, dtype=jnp.float32)[:, None, None], (b, TOPN, 1))
    pad = jnp.zeros((b, TOPN, 2), dtype=jnp.float32)
    output = jnp.concatenate([bidx, topv[..., None], pad, boxes],
                             axis=-1).reshape(b * TOPN, 8)
    return (output, loss[0, 0].reshape(()))
